# Initial kernel scaffold; baseline (speedup 1.0000x reference)
import jax
import jax.numpy as jnp
from jax.experimental import pallas as pl


def kernel(x, edge_index, eps, W1, b1, W2, b2):
    raise NotImplementedError("write your pallas kernel here")


# speedup vs baseline: 6.3110x; 6.3110x over previous
import functools

import jax
import jax.numpy as jnp
from jax import lax
from jax.experimental import pallas as pl
from jax.experimental.pallas import tpu as pltpu
from jax.experimental.pallas import tpu_sc as plsc

N = 10000
D = 128
E = 320000

NC = 2
NS = 16
EPC = E // NC
EPW = EPC // NS
CHUNK = 128
NFULL = EPW // CHUNK
TAIL = EPW - NFULL * CHUNK
N_PAD = 10240
RPT = N_PAD // NS
RPT_PIECES = (128, 128, 128, 128, 128)


def _sc_aggregate(x, src, dst):
    mesh = plsc.VectorSubcoreMesh(core_axis_name="c", subcore_axis_name="s")

    @functools.partial(
        pl.kernel,
        out_type=jax.ShapeDtypeStruct((NC, N_PAD, D), jnp.float32),
        mesh=mesh,
        scratch_types=[
            pltpu.VMEM((CHUNK,), jnp.int32),
            pltpu.VMEM((CHUNK,), jnp.int32),
            pltpu.VMEM((TAIL,), jnp.int32),
            pltpu.VMEM((TAIL,), jnp.int32),
            pltpu.VMEM((CHUNK, D), jnp.float32),
            pltpu.VMEM((TAIL, D), jnp.float32),
            pltpu.VMEM_SHARED((N_PAD, D), jnp.float32),
            pltpu.SemaphoreType.DMA,
        ],
    )
    def body(x_hbm, src_hbm, dst_hbm, out_hbm, src_v, dst_v, src_t, dst_t,
             rows_v, rows_t, acc, sem):
        c = lax.axis_index("c")
        s = lax.axis_index("s")

        z = jnp.zeros((16,), jnp.float32)

        def zero_row(i, carry):
            for k in range(D // 16):
                rows_v[i, pl.ds(k * 16, 16)] = z
            return carry

        lax.fori_loop(0, CHUNK, zero_row, 0)
        row0 = s * RPT
        off = 0
        for sz in RPT_PIECES:
            pltpu.sync_copy(rows_v.at[pl.ds(0, sz)],
                            acc.at[pl.ds(row0 + off, sz)])
            off += sz
        plsc.subcore_barrier()

        base = c * EPC + s * EPW

        def chunk_body(j, carry):
            off_e = base + j * CHUNK
            pltpu.sync_copy(src_hbm.at[pl.ds(off_e, CHUNK)], src_v)
            pltpu.sync_copy(dst_hbm.at[pl.ds(off_e, CHUNK)], dst_v)
            pltpu.async_copy(x_hbm.at[src_v], rows_v, sem).wait()
            pltpu.sync_copy(rows_v, acc.at[dst_v], add=True)
            return carry

        lax.fori_loop(0, NFULL, chunk_body, 0)

        off_e = base + NFULL * CHUNK
        pltpu.sync_copy(src_hbm.at[pl.ds(off_e, TAIL)], src_t)
        pltpu.sync_copy(dst_hbm.at[pl.ds(off_e, TAIL)], dst_t)
        pltpu.async_copy(x_hbm.at[src_t], rows_t, sem).wait()
        pltpu.sync_copy(rows_t, acc.at[dst_t], add=True)

        plsc.subcore_barrier()

        off = 0
        for sz in RPT_PIECES:
            pltpu.sync_copy(acc.at[pl.ds(row0 + off, sz)],
                            rows_v.at[pl.ds(0, sz)])
            pltpu.sync_copy(rows_v.at[pl.ds(0, sz)],
                            out_hbm.at[c, pl.ds(row0 + off, sz)])
            off += sz

    return body(x, src, dst)


def _mlp_body(x_ref, agg_ref, eps_ref, w1_ref, b1_ref, w2_ref, b2_ref,
              out_ref):
    h = x_ref[...] * eps_ref[0, 0] + agg_ref[0] + agg_ref[1]
    h = jnp.dot(h, w1_ref[...], preferred_element_type=jnp.float32,
                precision=lax.Precision.HIGHEST) + b1_ref[...]
    h = jnp.maximum(h, 0.0)
    out_ref[...] = jnp.dot(h, w2_ref[...], preferred_element_type=jnp.float32,
                           precision=lax.Precision.HIGHEST) + b2_ref[...]


def _mlp(x, agg2, eps, W1, b1, W2, b2):
    BR = 1000
    return pl.pallas_call(
        _mlp_body,
        grid=(N // BR,),
        in_specs=[
            pl.BlockSpec((BR, D), lambda i: (i, 0)),
            pl.BlockSpec((NC, BR, D), lambda i: (0, i, 0)),
            pl.BlockSpec(memory_space=pltpu.SMEM),
            pl.BlockSpec((D, D), lambda i: (0, 0)),
            pl.BlockSpec((1, D), lambda i: (0, 0)),
            pl.BlockSpec((D, D), lambda i: (0, 0)),
            pl.BlockSpec((1, D), lambda i: (0, 0)),
        ],
        out_specs=pl.BlockSpec((BR, D), lambda i: (i, 0)),
        out_shape=jax.ShapeDtypeStruct((N, D), jnp.float32),
    )(x, agg2, eps.reshape(1, 1), W1, b1.reshape(1, D), W2, b2.reshape(1, D))


def kernel(x, edge_index, eps, W1, b1, W2, b2):
    agg2 = _sc_aggregate(x, edge_index[0], edge_index[1])
    return _mlp(x, agg2, eps, W1, b1, W2, b2)
